# Initial kernel scaffold; baseline (speedup 1.0000x reference)
import jax
import jax.numpy as jnp
from jax.experimental import pallas as pl


def kernel(full_seq, table):
    raise NotImplementedError("write your pallas kernel here")


# speedup vs baseline: 1.2749x; 1.2749x over previous
import functools

import jax
import jax.numpy as jnp
import numpy as np
from jax import lax
from jax.experimental import pallas as pl
from jax.experimental.pallas import tpu as pltpu
from jax.experimental.pallas import tpu_sc as plsc

EMBED = 32
ROWS = 4096 * 200
NC, NS, L = 2, 16, 16
NW = NC * NS
ROWS_PER_W = ROWS // NW
CHUNK = 128
N_CHUNKS = ROWS_PER_W // CHUNK


def _embed_body(idx_hbm, table_hbm, out_hbm, idx_v, rows_v, sem):
    wid = lax.axis_index("s") * NC + lax.axis_index("c")
    base = wid * ROWS_PER_W

    def chunk_body(g, carry):
        off = base + g * CHUNK
        pltpu.sync_copy(idx_hbm.at[pl.ds(off, CHUNK)],
                        idx_v.at[pl.ds(0, CHUNK)])
        pltpu.async_copy(table_hbm.at[idx_v.at[pl.ds(0, CHUNK)]], rows_v,
                         sem).wait()

        m_acc = idx_v[pl.ds(0, L)] == jnp.int32(0)
        for i in range(1, CHUNK // L):
            m_acc = m_acc | (idx_v[pl.ds(i * L, L)] == jnp.int32(0))
        mi = jnp.where(m_acc, jnp.int32(1), jnp.int32(0))
        dnums = lax.GatherDimensionNumbers(
            offset_dims=(), collapsed_slice_dims=(0,), start_index_map=(0,))
        for k in (1, 2, 4, 8):
            perm = (lax.iota(jnp.int32, L) ^ jnp.int32(k)).reshape(L, 1)
            shuf = lax.gather(mi, perm, dnums, slice_sizes=(1,),
                              mode=lax.GatherScatterMode.PROMISE_IN_BOUNDS)
            mi = mi | shuf
        anyz = mi[0] > 0

        @pl.when(anyz)
        def _fix():
            def fix_row(r, c):
                s = idx_v[pl.ds(r, L)][0]
                @pl.when(s == jnp.int32(0))
                def _zero():
                    z = jnp.zeros((L,), jnp.float32)
                    rows_v[r, pl.ds(0, L)] = z
                    rows_v[r, pl.ds(L, L)] = z
                return c
            lax.fori_loop(0, CHUNK, fix_row, 0)

        pltpu.sync_copy(rows_v, out_hbm.at[pl.ds(off, CHUNK)])
        return carry

    lax.fori_loop(0, N_CHUNKS, chunk_body, 0)


@functools.partial(jax.jit, donate_argnums=())
def _embed_call(idx, table):
    mesh = plsc.VectorSubcoreMesh(core_axis_name="c", subcore_axis_name="s")
    fn = functools.partial(
        pl.kernel,
        mesh=mesh,
        compiler_params=pltpu.CompilerParams(use_tc_tiling_on_sc=False),
        out_type=jax.ShapeDtypeStruct((ROWS, EMBED), jnp.float32),
        scratch_types=[
            pltpu.VMEM((CHUNK + L,), jnp.int32),
            pltpu.VMEM((CHUNK, EMBED), jnp.float32),
            pltpu.SemaphoreType.DMA,
        ],
    )(_embed_body)
    return fn(idx, table)


def kernel(full_seq, table):
    idx = full_seq.reshape(-1).astype(jnp.int32)
    out = _embed_call(idx, table)
    return out.reshape(full_seq.shape[0], full_seq.shape[1], EMBED)


# speedup vs baseline: 1.5564x; 1.2208x over previous
import functools

import jax
import jax.numpy as jnp
from jax import lax
from jax.experimental import pallas as pl
from jax.experimental.pallas import tpu as pltpu
from jax.experimental.pallas import tpu_sc as plsc

EMBED = 32
ROWS = 4096 * 200
NC, NS, L = 2, 16, 16
NW = NC * NS
ROWS_PER_W = ROWS // NW
CHUNK = 128
GPB = 10
BLOCK = CHUNK * GPB
NBLK = ROWS_PER_W // BLOCK
NT = NBLK // 2


def _fire_gathers(table_hbm, idxb, rowsb, gsem):
    for j in range(GPB):
        pltpu.async_copy(
            table_hbm.at[idxb.at[pl.ds(j * CHUNK, CHUNK)]],
            rowsb.at[pl.ds(j * CHUNK, CHUNK)], gsem)


def _drain(src, dst, sem):
    pltpu.make_async_copy(src, dst, sem).wait()


def _fix_zero_rows(idxb, rowsb):
    m_acc = idxb[pl.ds(0, L)] == jnp.int32(0)
    for i in range(1, BLOCK // L):
        m_acc = m_acc | (idxb[pl.ds(i * L, L)] == jnp.int32(0))
    mi = jnp.where(m_acc, jnp.int32(1), jnp.int32(0))
    dnums = lax.GatherDimensionNumbers(
        offset_dims=(), collapsed_slice_dims=(0,), start_index_map=(0,))
    for k in (1, 2, 4, 8):
        perm = (lax.iota(jnp.int32, L) ^ jnp.int32(k)).reshape(L, 1)
        mi = mi | lax.gather(mi, perm, dnums, slice_sizes=(1,),
                             mode=lax.GatherScatterMode.PROMISE_IN_BOUNDS)

    @pl.when(mi[0] > 0)
    def _fix():
        def fix_row(r, c):
            s = idxb[pl.ds(r, L)][0]

            @pl.when(s == jnp.int32(0))
            def _zero():
                z = jnp.zeros((L,), jnp.float32)
                rowsb[r, pl.ds(0, L)] = z
                rowsb[r, pl.ds(L, L)] = z
            return c
        lax.fori_loop(0, BLOCK, fix_row, 0)


def _embed_body(idx_hbm, table_hbm, out_hbm,
                idx0, idx1, rows0, rows1, gsem, isem, ssem):
    wid = lax.axis_index("s") * NC + lax.axis_index("c")
    base = wid * ROWS_PER_W

    def idx_src(b):
        return idx_hbm.at[pl.ds(base + b * BLOCK, BLOCK)]

    def out_dst(b):
        return out_hbm.at[pl.ds(base + b * BLOCK, BLOCK)]

    pltpu.sync_copy(idx_src(0), idx0.at[pl.ds(0, BLOCK)])
    pltpu.async_copy(idx_src(1), idx1.at[pl.ds(0, BLOCK)], isem)
    _fire_gathers(table_hbm, idx0, rows0, gsem)

    def step(t, carry):
        a = 2 * t
        b = a + 1
        not_last = t < NT - 1

        _drain(idx_src(0), idx1.at[pl.ds(0, BLOCK)], isem)

        @pl.when(t > 0)
        def _():
            _drain(rows1, out_dst(0), ssem)
        _fire_gathers(table_hbm, idx1, rows1, gsem)

        _drain(out_dst(0), rows0, gsem)
        _fix_zero_rows(idx0, rows0)

        @pl.when(not_last)
        def _():
            pltpu.async_copy(idx_src(a + 2), idx0.at[pl.ds(0, BLOCK)], isem)
        pltpu.async_copy(rows0, out_dst(a), ssem)

        @pl.when(not_last)
        def _():
            _drain(idx_src(0), idx0.at[pl.ds(0, BLOCK)], isem)
        _drain(rows0, out_dst(0), ssem)

        @pl.when(not_last)
        def _():
            _fire_gathers(table_hbm, idx0, rows0, gsem)

        _drain(out_dst(0), rows1, gsem)
        _fix_zero_rows(idx1, rows1)

        @pl.when(not_last)
        def _():
            pltpu.async_copy(idx_src(b + 2), idx1.at[pl.ds(0, BLOCK)], isem)
        pltpu.async_copy(rows1, out_dst(b), ssem)
        return carry

    lax.fori_loop(0, NT, step, 0)
    _drain(rows1, out_dst(0), ssem)


@functools.partial(jax.jit, donate_argnums=())
def _embed_call(idx, table):
    mesh = plsc.VectorSubcoreMesh(core_axis_name="c", subcore_axis_name="s")
    fn = functools.partial(
        pl.kernel,
        mesh=mesh,
        compiler_params=pltpu.CompilerParams(use_tc_tiling_on_sc=False),
        out_type=jax.ShapeDtypeStruct((ROWS, EMBED), jnp.float32),
        scratch_types=[
            pltpu.VMEM((BLOCK + L,), jnp.int32),
            pltpu.VMEM((BLOCK + L,), jnp.int32),
            pltpu.VMEM((BLOCK, EMBED), jnp.float32),
            pltpu.VMEM((BLOCK, EMBED), jnp.float32),
            pltpu.SemaphoreType.DMA,
            pltpu.SemaphoreType.DMA,
            pltpu.SemaphoreType.DMA,
        ],
    )(_embed_body)
    return fn(idx, table)


def kernel(full_seq, table):
    idx = full_seq.reshape(-1).astype(jnp.int32)
    out = _embed_call(idx, table)
    return out.reshape(full_seq.shape[0], full_seq.shape[1], EMBED)
